# baseline (speedup 1.0000x reference)
import functools

import jax
import jax.numpy as jnp
from jax import lax
from jax.experimental import pallas as pl
from jax.experimental.pallas import tpu as pltpu
from jax.experimental.pallas import tpu_sc as plsc

_GATHER_W = 128
_VB = 8192


def _tc_transpose(tabT, T, V, D):
    Q = _VB // 4
    nvb = (V + _VB - 1) // _VB

    def body(in_ref, o_ref):
        x = in_ref[0]
        i = pl.program_id(1)
        valid = V - i * _VB
        lanes = lax.broadcasted_iota(jnp.int32, (D, _VB), 1)
        x = jnp.where(lanes < valid, x, 0.0)
        xt = jnp.transpose(x, (1, 0))
        for j in range(4):
            o_ref[0, :, j * D:(j + 1) * D] = xt[j * Q:(j + 1) * Q]

    return pl.pallas_call(
        body,
        grid=(T, nvb),
        in_specs=[pl.BlockSpec((1, D, _VB), lambda t, i: (t, 0, i))],
        out_specs=pl.BlockSpec((1, Q, 4 * D), lambda t, i: (t, i, 0)),
        out_shape=jax.ShapeDtypeStruct((T, nvb * Q, 4 * D), jnp.float32),
        compiler_params=pltpu.CompilerParams(
            dimension_semantics=("parallel", "arbitrary")),
    )(tabT)


def _sc_gather(table4, idx3, B, T, DW):
    mesh = plsc.VectorSubcoreMesh(core_axis_name="c", subcore_axis_name="s")

    @functools.partial(
        pl.kernel,
        out_type=jax.ShapeDtypeStruct((T, B, DW), table4.dtype),
        mesh=mesh,
    )
    def k(table_hbm, idx_hbm, out_hbm):
        def body(i_vmem, o_vmem):
            pltpu.sync_copy(table_hbm.at[i_vmem.at[0, 0]], o_vmem.at[0])

        pltpu.emit_pipeline(
            body,
            grid=(T, B // _GATHER_W),
            in_specs=[pl.BlockSpec((1, 1, _GATHER_W), lambda t, i: (t, 0, i))],
            out_specs=[pl.BlockSpec((1, _GATHER_W, DW), lambda t, i: (t, i, 0))],
            core_axis_name=("c", "s"),
            dimension_semantics=(pltpu.PARALLEL, pltpu.PARALLEL),
        )(idx_hbm, out_hbm)

    return k(table4, idx3)


def _matT(x, w):
    return lax.dot_general(x, w, (((1,), (1,)), ((), ())),
                           preferred_element_type=jnp.float32)


def _bottom_body(d_ref, w0, b0, w1, b1, w2, b2, o_ref):
    x = jnp.maximum(_matT(d_ref[...], w0[...]) + b0[...], 0.0)
    x = jnp.maximum(_matT(x.astype(jnp.bfloat16), w1[...]) + b1[...], 0.0)
    o_ref[...] = jnp.maximum(
        _matT(x.astype(jnp.bfloat16), w2[...]) + b2[...], 0.0
    ).astype(jnp.bfloat16)


def _full_spec(shape):
    ndims = len(shape)
    return pl.BlockSpec(shape, lambda *_args, _n=ndims: (0,) * _n)


def _bottom_mlp(dense, bw0, bb0, bw1, bb1, bw2, bb2, bm):
    B, F = dense.shape
    return pl.pallas_call(
        _bottom_body,
        grid=(B // bm,),
        in_specs=[
            pl.BlockSpec((bm, F), lambda i: (i, 0)),
            _full_spec(bw0.shape), _full_spec(bb0.shape),
            _full_spec(bw1.shape), _full_spec(bb1.shape),
            _full_spec(bw2.shape), _full_spec(bb2.shape),
        ],
        out_specs=pl.BlockSpec((bm, bw2.shape[0]), lambda i: (i, 0)),
        out_shape=jax.ShapeDtypeStruct((B, bw2.shape[0]), jnp.bfloat16),
        compiler_params=pltpu.CompilerParams(
            dimension_semantics=("parallel",)),
    )(dense, bw0, bb0, bw1, bb1, bw2, bb2)


def _top_mlp(x, ly_chunks, sel, tw0x, tw0e, tb0, tw1, tb1, tw2, tb2, D, bm):
    B = x.shape[0]
    nch = len(ly_chunks)

    def body(*refs):
        x_ref = refs[0]
        ly_refs = refs[1:1 + nch]
        sel_ref, w0x, w0e, b0, w1, b1, w2, b2, o_ref = refs[1 + nch:]
        s2 = sel_ref[...]
        lane = (lax.broadcasted_iota(jnp.int32, (1, 4 * D), 1)
                // D).astype(jnp.float32)
        pieces = []
        t = 0
        for ly_ref in ly_refs:
            big = ly_ref[...]
            for tc in range(big.shape[0]):
                st = s2[t][:, None]
                pieces.append((big[tc] * (st == lane).astype(jnp.float32)
                               ).astype(jnp.bfloat16))
                t += 1
        z_e = jnp.concatenate(pieces, axis=1)
        a = _matT(x_ref[...], w0x[...]) + _matT(z_e, w0e[...]) + b0[...]
        z = jnp.maximum(a, 0.0).astype(jnp.bfloat16)
        z = jnp.maximum(_matT(z, w1[...]) + b1[...], 0.0).astype(jnp.bfloat16)
        r = _matT(z, w2[...])
        o_ref[...] = jax.nn.sigmoid(r[:, :1] + b2[0, 0])

    T = sel.shape[0]
    return pl.pallas_call(
        body,
        grid=(B // bm,),
        in_specs=[
            pl.BlockSpec((bm, x.shape[1]), lambda i: (i, 0)),
            *[pl.BlockSpec((c.shape[0], bm, c.shape[2]), lambda i: (0, i, 0))
              for c in ly_chunks],
            pl.BlockSpec((T, bm), lambda i: (0, i)),
            _full_spec(tw0x.shape), _full_spec(tw0e.shape),
            _full_spec(tb0.shape),
            _full_spec(tw1.shape), _full_spec(tb1.shape),
            _full_spec(tw2.shape), _full_spec(tb2.shape),
        ],
        out_specs=pl.BlockSpec((bm, 1), lambda i: (i, 0)),
        out_shape=jax.ShapeDtypeStruct((B, 1), jnp.float32),
        compiler_params=pltpu.CompilerParams(
            dimension_semantics=("parallel",)),
    )(x, *ly_chunks, sel, tw0x, tw0e, tb0, tw1, tb1, tw2, tb2)


def kernel(dense_input, emb_tables, bw0, bb0, bw1, bb1, bw2, bb2,
           tw0, tb0, tw1, tb1, tw2, tb2, indices, offsets):
    del offsets
    T, V, D = emb_tables.shape
    B = dense_input.shape[0]

    tabT = jnp.swapaxes(emb_tables, 1, 2)

    Q = _VB // 4
    sel = ((indices % _VB) // Q).astype(jnp.float32)

    bounds = [0, 7, 14, 20, 26] if T == 26 else list(range(T + 1))
    ly_chunks = []
    for t0, t1 in zip(bounds[:-1], bounds[1:]):
        Tc = t1 - t0
        t4c = _tc_transpose(tabT[t0:t1], Tc, V, D)
        rows_per_t = t4c.shape[1]
        tabflat = t4c.reshape(Tc * rows_per_t, 4 * D)
        basec = (jnp.arange(Tc, dtype=jnp.int32) * rows_per_t)[:, None]
        idxc = (basec + (indices[t0:t1] // _VB) * Q
                + (indices[t0:t1] % Q)).reshape(Tc, 1, B)
        ly_chunks.append(_sc_gather(tabflat, idxc, B, Tc, 4 * D))

    bf = jnp.bfloat16
    x = _bottom_mlp(dense_input.astype(bf), bw0.astype(bf), bb0[None, :],
                    bw1.astype(bf), bb1[None, :], bw2.astype(bf),
                    bb2[None, :], 2048)

    nbot = bw2.shape[0]
    n1 = tw0.shape[0]
    tw0x = tw0[:, :nbot].astype(bf)
    tw0e = jnp.tile(tw0[:, nbot:].reshape(n1, T, 1, D),
                    (1, 1, 4, 1)).reshape(n1, T * 4 * D).astype(bf)
    tw2p = jnp.zeros((128, tw2.shape[1]), bf).at[:1].set(tw2.astype(bf))
    return _top_mlp(x, ly_chunks, sel, tw0x, tw0e, tb0[None, :],
                    tw1.astype(bf), tb1[None, :],
                    tw2p, tb2[None, :], D, 512)


# speedup vs baseline: 1.1183x; 1.1183x over previous
import functools

import jax
import jax.numpy as jnp
from jax import lax
from jax.experimental import pallas as pl
from jax.experimental.pallas import tpu as pltpu
from jax.experimental.pallas import tpu_sc as plsc

_GATHER_W = 128
_VB = 8192


def _tc_transpose(tabT, T, V, D):
    Q = _VB // 4
    nvb = (V + _VB - 1) // _VB

    def body(in_ref, o_ref):
        x = in_ref[0]
        i = pl.program_id(1)
        valid = V - i * _VB
        lanes = lax.broadcasted_iota(jnp.int32, (D, _VB), 1)
        x = jnp.where(lanes < valid, x, 0.0)
        xt = jnp.transpose(x, (1, 0))
        for j in range(4):
            o_ref[0, :, j * D:(j + 1) * D] = xt[j * Q:(j + 1) * Q]

    return pl.pallas_call(
        body,
        grid=(T, nvb),
        in_specs=[pl.BlockSpec((1, D, _VB), lambda t, i: (t, 0, i))],
        out_specs=pl.BlockSpec((1, Q, 4 * D), lambda t, i: (t, i, 0)),
        out_shape=jax.ShapeDtypeStruct((T, nvb * Q, 4 * D), jnp.float32),
        compiler_params=pltpu.CompilerParams(
            dimension_semantics=("parallel", "arbitrary")),
    )(tabT)


def _sc_gather(table4, idx3, B, T, DW):
    mesh = plsc.VectorSubcoreMesh(core_axis_name="c", subcore_axis_name="s")

    @functools.partial(
        pl.kernel,
        out_type=jax.ShapeDtypeStruct((T, B, DW), table4.dtype),
        mesh=mesh,
    )
    def k(table_hbm, idx_hbm, out_hbm):
        def body(i_vmem, o_vmem):
            pltpu.sync_copy(table_hbm.at[i_vmem.at[0, 0]], o_vmem.at[0])

        pltpu.emit_pipeline(
            body,
            grid=(T, B // _GATHER_W),
            in_specs=[pl.BlockSpec((1, 1, _GATHER_W), lambda t, i: (t, 0, i))],
            out_specs=[pl.BlockSpec((1, _GATHER_W, DW), lambda t, i: (t, i, 0))],
            core_axis_name=("c", "s"),
            dimension_semantics=(pltpu.PARALLEL, pltpu.PARALLEL),
        )(idx_hbm, out_hbm)

    return k(table4, idx3)


def _matT(x, w):
    return lax.dot_general(x, w, (((1,), (1,)), ((), ())),
                           preferred_element_type=jnp.float32)


def _bottom_body(d_ref, w0, b0, w1, b1, w2, b2, o_ref):
    x = jnp.maximum(_matT(d_ref[...], w0[...]) + b0[...], 0.0)
    x = jnp.maximum(_matT(x.astype(jnp.bfloat16), w1[...]) + b1[...], 0.0)
    o_ref[...] = jnp.maximum(
        _matT(x.astype(jnp.bfloat16), w2[...]) + b2[...], 0.0
    ).astype(jnp.bfloat16)


def _full_spec(shape):
    ndims = len(shape)
    return pl.BlockSpec(shape, lambda *_args, _n=ndims: (0,) * _n)


def _bottom_mlp(dense, bw0, bb0, bw1, bb1, bw2, bb2, bm):
    B, F = dense.shape
    return pl.pallas_call(
        _bottom_body,
        grid=(B // bm,),
        in_specs=[
            pl.BlockSpec((bm, F), lambda i: (i, 0)),
            _full_spec(bw0.shape), _full_spec(bb0.shape),
            _full_spec(bw1.shape), _full_spec(bb1.shape),
            _full_spec(bw2.shape), _full_spec(bb2.shape),
        ],
        out_specs=pl.BlockSpec((bm, bw2.shape[0]), lambda i: (i, 0)),
        out_shape=jax.ShapeDtypeStruct((B, bw2.shape[0]), jnp.bfloat16),
        compiler_params=pltpu.CompilerParams(
            dimension_semantics=("parallel",)),
    )(dense, bw0, bb0, bw1, bb1, bw2, bb2)


def _top_mlp(x, ly_chunks, sel, tw0x, tw0e, tb0, tw1, tb1, tw2, tb2, D, bm):
    B = x.shape[0]
    nch = len(ly_chunks)

    def body(*refs):
        x_ref = refs[0]
        ly_refs = refs[1:1 + nch]
        sel_ref, w0x, w0e, b0, w1, b1, w2, b2, o_ref = refs[1 + nch:]
        s2 = sel_ref[...]
        lane = (lax.broadcasted_iota(jnp.int32, (1, 4 * D), 1)
                // D).astype(jnp.float32)
        pieces = []
        t = 0
        for ly_ref in ly_refs:
            big = ly_ref[...]
            for tc in range(big.shape[0]):
                st = s2[t][:, None]
                pieces.append((big[tc] * (st == lane).astype(jnp.float32)
                               ).astype(jnp.bfloat16))
                t += 1
        z_e = jnp.concatenate(pieces, axis=1)
        a = _matT(x_ref[...], w0x[...]) + _matT(z_e, w0e[...]) + b0[...]
        z = jnp.maximum(a, 0.0).astype(jnp.bfloat16)
        z = jnp.maximum(_matT(z, w1[...]) + b1[...], 0.0).astype(jnp.bfloat16)
        r = _matT(z, w2[...])
        o_ref[...] = jax.nn.sigmoid(r[:, :1] + b2[0, 0])

    T = sel.shape[0]
    return pl.pallas_call(
        body,
        grid=(B // bm,),
        in_specs=[
            pl.BlockSpec((bm, x.shape[1]), lambda i: (i, 0)),
            *[pl.BlockSpec((c.shape[0], bm, c.shape[2]), lambda i: (0, i, 0))
              for c in ly_chunks],
            pl.BlockSpec((T, bm), lambda i: (0, i)),
            _full_spec(tw0x.shape), _full_spec(tw0e.shape),
            _full_spec(tb0.shape),
            _full_spec(tw1.shape), _full_spec(tb1.shape),
            _full_spec(tw2.shape), _full_spec(tb2.shape),
        ],
        out_specs=pl.BlockSpec((bm, 1), lambda i: (i, 0)),
        out_shape=jax.ShapeDtypeStruct((B, 1), jnp.float32),
        compiler_params=pltpu.CompilerParams(
            dimension_semantics=("parallel",)),
    )(x, *ly_chunks, sel, tw0x, tw0e, tb0, tw1, tb1, tw2, tb2)


def kernel(dense_input, emb_tables, bw0, bb0, bw1, bb1, bw2, bb2,
           tw0, tb0, tw1, tb1, tw2, tb2, indices, offsets):
    del offsets
    T, V, D = emb_tables.shape
    B = dense_input.shape[0]

    tabT = jnp.swapaxes(emb_tables, 1, 2)

    Q = _VB // 4
    sel = ((indices % _VB) // Q).astype(jnp.float32)

    bounds = [0, T]
    ly_chunks = []
    for t0, t1 in zip(bounds[:-1], bounds[1:]):
        Tc = t1 - t0
        t4c = _tc_transpose(tabT[t0:t1], Tc, V, D)
        rows_per_t = t4c.shape[1]
        tabflat = t4c.reshape(Tc * rows_per_t, 4 * D)
        basec = (jnp.arange(Tc, dtype=jnp.int32) * rows_per_t)[:, None]
        idxc = (basec + (indices[t0:t1] // _VB) * Q
                + (indices[t0:t1] % Q)).reshape(Tc, 1, B)
        ly_chunks.append(_sc_gather(tabflat, idxc, B, Tc, 4 * D))

    bf = jnp.bfloat16
    x = _bottom_mlp(dense_input.astype(bf), bw0.astype(bf), bb0[None, :],
                    bw1.astype(bf), bb1[None, :], bw2.astype(bf),
                    bb2[None, :], 2048)

    nbot = bw2.shape[0]
    n1 = tw0.shape[0]
    tw0x = tw0[:, :nbot].astype(bf)
    tw0e = jnp.tile(tw0[:, nbot:].reshape(n1, T, 1, D),
                    (1, 1, 4, 1)).reshape(n1, T * 4 * D).astype(bf)
    tw2p = jnp.zeros((128, tw2.shape[1]), bf).at[:1].set(tw2.astype(bf))
    return _top_mlp(x, ly_chunks, sel, tw0x, tw0e, tb0[None, :],
                    tw1.astype(bf), tb1[None, :],
                    tw2p, tb2[None, :], D, 512)
